# baseline (speedup 1.0000x reference)
import functools

import jax
import jax.numpy as jnp
from jax.experimental import pallas as pl
from jax.experimental.pallas import tpu as pltpu

B = 32
V = 128
C = 4
G = 4
NSTEPS = B // G

_F32 = jnp.float32


def _dot(x, y, dims):
    return jax.lax.dot_general(x, y, (dims, ((), ())),
                               preferred_element_type=_F32)


_NN = (((1,), (0,)))
_NT = (((1,), (1,)))
_TT = (((0,), (1,)))
_TN = (((0,), (0,)))


def _fused_kernel(edges_ref, nodesT_ref, nparamsT_ref, condT_ref,
                  ne0_Wt, ne0_b, ne1_W, ne1_b, g1_W, g1_b, g2_Wt, g2_b,
                  g3a_W, g3a_b, g3b_W, g3b_b, ce0_Wt, ce0_b, ce1_W, ce1_b,
                  fc0_Wt, fc0_b, fc1_W, fc1_b, fc2_W, fc2_b, fc3_W, fc3_b,
                  out_ref, mu_acc):
    (ne0_Wt, ne0_b, ne1_W, ne1_b, g1_W, g1_b, g2_Wt, g2_b) = (
        r[...] for r in (
            ne0_Wt, ne0_b, ne1_W, ne1_b, g1_W, g1_b, g2_Wt, g2_b))
    step = pl.program_id(0)

    i_idx = jax.lax.broadcasted_iota(jnp.int32, (V, V * C), 0)
    k_idx = jax.lax.broadcasted_iota(jnp.int32, (V, V * C), 1)
    Msel = jnp.where((k_idx // C == i_idx) & (k_idx % C != 0),
                     _F32(1.0), _F32(0.0))

    W_ne = _dot(ne1_W, ne0_Wt, _NT)
    b_ne = _dot(ne0_b, ne1_W, _NT) + ne1_b
    g1n = g1_W[:, :8]
    g1h = g1_W[:, 8:]
    Wp = _dot(g1h, W_ne, _NN)
    b0 = _dot(b_ne, g1h, _NT) + g1_b

    As, degs, n0s = [], [], []
    for b in range(G):
        Eb = edges_ref[b * V * C:(b + 1) * V * C, :]
        Ab = _dot(Msel, Eb, _NN)
        As.append(Ab)
        degs.append(jnp.sum(Ab, axis=1, keepdims=True) + 1e-8)
        Xn = nodesT_ref[b * 8:(b + 1) * 8, :]
        Xp = nparamsT_ref[b * 16:(b + 1) * 16, :]
        n0s.append(_dot(Xn, g1n, _TT) + _dot(Xp, Wp, _TT) + b0)

    m1_parts = [
        _dot(As[b], n0s[b], _NN) / degs[b] for b in range(G)]
    m1 = jnp.concatenate(m1_parts, axis=0)

    n1 = _dot(m1, g2_Wt, _NN) + g2_b

    mus = []
    for b in range(G):
        s = slice(b * V, (b + 1) * V)
        degb = degs[b]
        m2 = _dot(As[b], n1[s], _NN) / degb
        d = m2 - jnp.mean(m2, axis=0, keepdims=True)
        ad = _dot(As[b], d, _NN)
        ev = ((degb - 1e-8) * d - ad) / degb
        mus.append(jnp.concatenate(
            [jnp.mean(m2, axis=0, keepdims=True),
             jnp.mean(ev, axis=0, keepdims=True)], axis=1))
    mu_acc[pl.ds(step * G, G), :] = jnp.concatenate(mus, axis=0)

    @pl.when(step == NSTEPS - 1)
    def _tail():
        (g3a_W_, g3a_b_, g3b_W_, g3b_b_, ce0_Wt_, ce0_b_, ce1_W_, ce1_b_,
         fc0_Wt_, fc0_b_, fc1_W_, fc1_b_, fc2_W_, fc2_b_, fc3_W_) = (
            r[...] for r in (
                g3a_W, g3a_b, g3b_W, g3b_b, ce0_Wt, ce0_b, ce1_W, ce1_b,
                fc0_Wt, fc0_b, fc1_W, fc1_b, fc2_W, fc2_b, fc3_W))
        mu = mu_acc[...]
        gl = _dot(mu, g3a_W_, _NT) + g3a_b_
        gl = _dot(gl, g3b_W_, _NT) + g3b_b_
        c = _dot(condT_ref[...], ce0_Wt_, _TN) + ce0_b_
        c = _dot(c, ce1_W_, _NT) + ce1_b_
        gl = jnp.concatenate([gl, c], axis=1)
        gl = _dot(gl, fc0_Wt_, _NN) + fc0_b_
        gl = _dot(gl, fc1_W_, _NT) + fc1_b_
        gl = _dot(gl, fc2_W_, _NT) + fc2_b_
        t = gl * fc3_W_
        s = _dot(t, jnp.ones((32, V), _F32), _NN)
        out_ref[...] = s + fc3_b[0, 0]


@functools.partial(jax.jit, static_argnames=())
def kernel(edges, hidden, nodes, node_params, cond,
           ne0_W, ne0_b, ne1_W, ne1_b, g1_W, g1_b, g2_W, g2_b,
           g3a_W, g3a_b, g3b_W, g3b_b, ce0_W, ce0_b, ce1_W, ce1_b,
           fc0_W, fc0_b, fc1_W, fc1_b, fc2_W, fc2_b, fc3_W, fc3_b):
    del hidden
    edges2d = edges.transpose(0, 1, 3, 2).reshape(B * V * C, V)
    nodesT = nodes.transpose(0, 2, 1).reshape(B * 8, V)
    nparamsT = node_params.transpose(0, 2, 1).reshape(B * 16, V)
    args = [edges2d, nodesT, nparamsT, cond.T,
            ne0_W.T, ne0_b.reshape(1, -1), ne1_W, ne1_b.reshape(1, -1),
            g1_W, g1_b.reshape(1, -1), g2_W.T, g2_b.reshape(1, -1),
            g3a_W, g3a_b.reshape(1, -1), g3b_W, g3b_b.reshape(1, -1),
            ce0_W.T, ce0_b.reshape(1, -1), ce1_W, ce1_b.reshape(1, -1),
            fc0_W.T, fc0_b.reshape(1, -1), fc1_W, fc1_b.reshape(1, -1),
            fc2_W, fc2_b.reshape(1, -1), fc3_W, fc3_b.reshape(1, -1)]
    full = lambda shape: pl.BlockSpec(shape, lambda i: (0, 0))
    out = pl.pallas_call(
        _fused_kernel,
        grid=(NSTEPS,),
        in_specs=[
            pl.BlockSpec((G * V * C, V), lambda i: (i, 0)),
            pl.BlockSpec((G * 8, V), lambda i: (i, 0)),
            pl.BlockSpec((G * 16, V), lambda i: (i, 0)),
            full((10, 32)),
            full((16, 64)), full((1, 64)),
            full((32, 64)), full((1, 32)),
            full((32, 40)), full((1, 32)),
            full((32, 64)), full((1, 64)),
            full((256, 128)), full((1, 256)),
            full((128, 256)), full((1, 128)),
            full((10, 32)), full((1, 32)),
            full((16, 32)), full((1, 16)),
            full((144, 128)), full((1, 128)),
            full((64, 128)), full((1, 64)),
            full((32, 64)), full((1, 32)),
            full((1, 32)), full((1, 1)),
        ],
        out_specs=full((B, V)),
        out_shape=jax.ShapeDtypeStruct((B, V), jnp.float32),
        scratch_shapes=[pltpu.VMEM((B, V), jnp.float32)],
    )(*args)
    return out[:, :1]


# speedup vs baseline: 1.1804x; 1.1804x over previous
import functools

import jax
import jax.numpy as jnp
from jax.experimental import pallas as pl
from jax.experimental.pallas import tpu as pltpu

B = 32
V = 128
C = 4
NG = 8
GR = B // NG
ROWS = GR * V * C

_F32 = jnp.float32


def _dot(x, y, dims):
    return jax.lax.dot_general(x, y, (dims, ((), ())),
                               preferred_element_type=_F32)


_NN = ((1,), (0,))
_NT = ((1,), (1,))
_TT = ((0,), (1,))
_TN = ((0,), (0,))


def _fused_kernel(edges_hbm, nodesT_ref, nparamsT_ref, condT_ref,
                  ne0_Wt, ne0_b, ne1_W, ne1_b, g1_W, g1_b, g2_Wt, g2_b,
                  g3a_W, g3a_b, g3b_W, g3b_b, ce0_Wt, ce0_b, ce1_W, ce1_b,
                  fc0_Wt, fc0_b, fc1_W, fc1_b, fc2_W, fc2_b, fc3_W, fc3_b,
                  out_ref, ebuf0, ebuf1, sems):
    (ne0_Wt, ne0_b, ne1_W, ne1_b, g1_W, g1_b, g2_Wt, g2_b,
     g3a_W, g3a_b, g3b_W, g3b_b, ce0_Wt, ce0_b, ce1_W, ce1_b,
     fc0_Wt, fc0_b, fc1_W, fc1_b, fc2_W, fc2_b, fc3_W) = (
        r[...] for r in (
            ne0_Wt, ne0_b, ne1_W, ne1_b, g1_W, g1_b, g2_Wt, g2_b,
            g3a_W, g3a_b, g3b_W, g3b_b, ce0_Wt, ce0_b, ce1_W, ce1_b,
            fc0_Wt, fc0_b, fc1_W, fc1_b, fc2_W, fc2_b, fc3_W))
    bufs = (ebuf0, ebuf1)

    def copy(g):
        return pltpu.make_async_copy(
            edges_hbm.at[pl.ds(g * ROWS, ROWS), :],
            bufs[g % 2], sems.at[g % 2])

    copy(0).start()

    W_ne = _dot(ne1_W, ne0_Wt, _NT)
    b_ne = _dot(ne0_b, ne1_W, _NT) + ne1_b
    g1n = g1_W[:, :8]
    g1h = g1_W[:, 8:]
    Wp = _dot(g1h, W_ne, _NN)
    b0 = _dot(b_ne, g1h, _NT) + g1_b

    n0s = []
    for b in range(B):
        Xn = nodesT_ref[b * 8:(b + 1) * 8, :]
        Xp = nparamsT_ref[b * 16:(b + 1) * 16, :]
        n0s.append(_dot(Xn, g1n, _TT) + _dot(Xp, Wp, _TT) + b0)

    mus = []
    for g in range(NG):
        if g + 1 < NG:
            copy(g + 1).start()
        copy(g).wait()
        ebuf = bufs[g % 2]
        A_grp = (ebuf[1::C, :] + ebuf[2::C, :]
                 + ebuf[3::C, :])
        deg_grp = jnp.sum(A_grp, axis=1, keepdims=True) + 1e-8

        m1_parts = []
        for j in range(GR):
            b = g * GR + j
            Ab = A_grp[j * V:(j + 1) * V, :]
            m1_parts.append(
                _dot(Ab, n0s[b], _NN) / deg_grp[j * V:(j + 1) * V, :])
        m1 = jnp.concatenate(m1_parts, axis=0)

        n1 = _dot(m1, g2_Wt, _NN) + g2_b

        for j in range(GR):
            s = slice(j * V, (j + 1) * V)
            Ab = A_grp[s, :]
            degb = deg_grp[s, :]
            m2 = _dot(Ab, n1[s], _NN) / degb
            d = m2 - jnp.mean(m2, axis=0, keepdims=True)
            ad = _dot(Ab, d, _NN)
            ev = ((degb - 1e-8) * d - ad) / degb
            mus.append(jnp.concatenate(
                [jnp.mean(m2, axis=0, keepdims=True),
                 jnp.mean(ev, axis=0, keepdims=True)], axis=1))
    mu = jnp.concatenate(mus, axis=0)

    gl = _dot(mu, g3a_W, _NT) + g3a_b
    gl = _dot(gl, g3b_W, _NT) + g3b_b
    c = _dot(condT_ref[...], ce0_Wt, _TN) + ce0_b
    c = _dot(c, ce1_W, _NT) + ce1_b
    gl = jnp.concatenate([gl, c], axis=1)
    gl = _dot(gl, fc0_Wt, _NN) + fc0_b
    gl = _dot(gl, fc1_W, _NT) + fc1_b
    gl = _dot(gl, fc2_W, _NT) + fc2_b
    t = gl * fc3_W
    s = _dot(t, jnp.ones((32, V), _F32), _NN)
    out_ref[...] = s + fc3_b[0, 0]


@functools.partial(jax.jit, static_argnames=())
def kernel(edges, hidden, nodes, node_params, cond,
           ne0_W, ne0_b, ne1_W, ne1_b, g1_W, g1_b, g2_W, g2_b,
           g3a_W, g3a_b, g3b_W, g3b_b, ce0_W, ce0_b, ce1_W, ce1_b,
           fc0_W, fc0_b, fc1_W, fc1_b, fc2_W, fc2_b, fc3_W, fc3_b):
    del hidden
    edges2d = edges.transpose(0, 1, 3, 2).reshape(B * V * C, V)
    nodesT = nodes.transpose(0, 2, 1).reshape(B * 8, V)
    nparamsT = node_params.transpose(0, 2, 1).reshape(B * 16, V)
    args = [edges2d, nodesT, nparamsT, cond.T,
            ne0_W.T, ne0_b.reshape(1, -1), ne1_W, ne1_b.reshape(1, -1),
            g1_W, g1_b.reshape(1, -1), g2_W.T, g2_b.reshape(1, -1),
            g3a_W, g3a_b.reshape(1, -1), g3b_W, g3b_b.reshape(1, -1),
            ce0_W.T, ce0_b.reshape(1, -1), ce1_W, ce1_b.reshape(1, -1),
            fc0_W.T, fc0_b.reshape(1, -1), fc1_W, fc1_b.reshape(1, -1),
            fc2_W, fc2_b.reshape(1, -1), fc3_W, fc3_b.reshape(1, -1)]
    in_specs = [pl.BlockSpec(memory_space=pltpu.MemorySpace.HBM)]
    in_specs += [pl.BlockSpec(memory_space=pltpu.MemorySpace.VMEM)
                 for _ in range(len(args) - 1)]
    out = pl.pallas_call(
        _fused_kernel,
        in_specs=in_specs,
        out_specs=pl.BlockSpec(memory_space=pltpu.MemorySpace.VMEM),
        out_shape=jax.ShapeDtypeStruct((B, V), jnp.float32),
        scratch_shapes=[pltpu.VMEM((ROWS, V), jnp.float32),
                        pltpu.VMEM((ROWS, V), jnp.float32),
                        pltpu.SemaphoreType.DMA((2,))],
    )(*args)
    return out[:, :1]


# speedup vs baseline: 1.5285x; 1.2949x over previous
import functools

import jax
import jax.numpy as jnp
from jax.experimental import pallas as pl

B = 32
V = 128
C = 4

_F32 = jnp.float32


def _dot(x, y, dims):
    return jax.lax.dot_general(x, y, (dims, ((), ())),
                               preferred_element_type=_F32)


_NN = ((1,), (0,))
_NT = ((1,), (1,))
_TT = ((0,), (1,))
_TN = ((0,), (0,))


def _fused_kernel(edges_ref, nodesT_ref, nparamsT_ref, condT_ref,
                  ne0_Wt, ne0_b, ne1_W, ne1_b, g1_W, g1_b, g2_Wt, g2_b,
                  g3a_W, g3a_b, g3b_W, g3b_b, ce0_Wt, ce0_b, ce1_W, ce1_b,
                  fc0_Wt, fc0_b, fc1_W, fc1_b, fc2_W, fc2_b, fc3_W, fc3_b,
                  out_ref):
    (ne0_Wt, ne0_b, ne1_W, ne1_b, g1_W, g1_b, g2_Wt, g2_b,
     g3a_W, g3a_b, g3b_W, g3b_b, ce0_Wt, ce0_b, ce1_W, ce1_b,
     fc0_Wt, fc0_b, fc1_W, fc1_b, fc2_W, fc2_b, fc3_W) = (
        r[...] for r in (
            ne0_Wt, ne0_b, ne1_W, ne1_b, g1_W, g1_b, g2_Wt, g2_b,
            g3a_W, g3a_b, g3b_W, g3b_b, ce0_Wt, ce0_b, ce1_W, ce1_b,
            fc0_Wt, fc0_b, fc1_W, fc1_b, fc2_W, fc2_b, fc3_W))
    W_ne = _dot(ne1_W, ne0_Wt, _NT)
    b_ne = _dot(ne0_b, ne1_W, _NT) + ne1_b
    g1n = g1_W[:, :8]
    g1h = g1_W[:, 8:]
    Wp = _dot(g1h, W_ne, _NN)
    b0 = _dot(b_ne, g1h, _NT) + g1_b

    A_all = (edges_ref[1::C, :] + edges_ref[2::C, :]
             + edges_ref[3::C, :])
    deg_all = jnp.sum(A_all, axis=1, keepdims=True) + 1e-8

    As = [A_all[b * V:(b + 1) * V, :] for b in range(B)]
    degs = [deg_all[b * V:(b + 1) * V, :] for b in range(B)]
    n0s = []
    for b in range(B):
        Xn = nodesT_ref[b * 8:(b + 1) * 8, :]
        Xp = nparamsT_ref[b * 16:(b + 1) * 16, :]
        n0s.append(_dot(Xn, g1n, _TT) + _dot(Xp, Wp, _TT) + b0)

    m1_parts = [
        _dot(As[b], n0s[b], _NN) / degs[b] for b in range(B)]
    m1 = jnp.concatenate(m1_parts, axis=0)

    n1 = _dot(m1, g2_Wt, _NN) + g2_b

    m2s = [_dot(As[b], n1[b * V:(b + 1) * V], _NN) / degs[b]
           for b in range(B)]
    mean_m2s = [jnp.mean(m2s[b], axis=0, keepdims=True) for b in range(B)]
    ds = [m2s[b] - mean_m2s[b] for b in range(B)]
    ads = [_dot(As[b], ds[b], _NN) for b in range(B)]
    mus = []
    for b in range(B):
        degb = degs[b]
        ev = ((degb - 1e-8) * ds[b] - ads[b]) / degb
        mus.append(jnp.concatenate(
            [mean_m2s[b],
             jnp.mean(ev, axis=0, keepdims=True)], axis=1))
    mu = jnp.concatenate(mus, axis=0)

    gl = _dot(mu, g3a_W, _NT) + g3a_b
    gl = _dot(gl, g3b_W, _NT) + g3b_b
    c = _dot(condT_ref[...], ce0_Wt, _TN) + ce0_b
    c = _dot(c, ce1_W, _NT) + ce1_b
    gl = jnp.concatenate([gl, c], axis=1)
    gl = _dot(gl, fc0_Wt, _NN) + fc0_b
    gl = _dot(gl, fc1_W, _NT) + fc1_b
    gl = _dot(gl, fc2_W, _NT) + fc2_b
    t = gl * fc3_W
    s = _dot(t, jnp.ones((32, V), _F32), _NN)
    out_ref[...] = s + fc3_b[0, 0]


@functools.partial(jax.jit, static_argnames=())
def kernel(edges, hidden, nodes, node_params, cond,
           ne0_W, ne0_b, ne1_W, ne1_b, g1_W, g1_b, g2_W, g2_b,
           g3a_W, g3a_b, g3b_W, g3b_b, ce0_W, ce0_b, ce1_W, ce1_b,
           fc0_W, fc0_b, fc1_W, fc1_b, fc2_W, fc2_b, fc3_W, fc3_b):
    del hidden
    edges2d = edges.transpose(0, 1, 3, 2).reshape(B * V * C, V)
    nodesT = nodes.transpose(0, 2, 1).reshape(B * 8, V)
    nparamsT = node_params.transpose(0, 2, 1).reshape(B * 16, V)
    args = [edges2d, nodesT, nparamsT, cond.T,
            ne0_W.T, ne0_b.reshape(1, -1), ne1_W, ne1_b.reshape(1, -1),
            g1_W, g1_b.reshape(1, -1), g2_W.T, g2_b.reshape(1, -1),
            g3a_W, g3a_b.reshape(1, -1), g3b_W, g3b_b.reshape(1, -1),
            ce0_W.T, ce0_b.reshape(1, -1), ce1_W, ce1_b.reshape(1, -1),
            fc0_W.T, fc0_b.reshape(1, -1), fc1_W, fc1_b.reshape(1, -1),
            fc2_W, fc2_b.reshape(1, -1), fc3_W, fc3_b.reshape(1, -1)]
    out = pl.pallas_call(
        _fused_kernel,
        out_shape=jax.ShapeDtypeStruct((B, V), jnp.float32),
    )(*args)
    return out[:, :1]


# speedup vs baseline: 1.6049x; 1.0500x over previous
import functools

import jax
import jax.numpy as jnp
from jax.experimental import pallas as pl
from jax.experimental.pallas import tpu as pltpu

B = 32
V = 128
C = 4
NG = 4
GR = B // NG
ROWS = GR * V * C

_F32 = jnp.float32


def _dot(x, y, dims):
    return jax.lax.dot_general(x, y, (dims, ((), ())),
                               preferred_element_type=_F32)


_NN = ((1,), (0,))
_NT = ((1,), (1,))
_TT = ((0,), (1,))
_TN = ((0,), (0,))


def _fused_kernel(edges_hbm, nodesT_ref, nparamsT_ref, condT_ref,
                  ne0_Wt, ne0_b, ne1_W, ne1_b, g1_W, g1_b, g2_Wt, g2_b,
                  g3a_W, g3a_b, g3b_W, g3b_b, ce0_Wt, ce0_b, ce1_W, ce1_b,
                  fc0_Wt, fc0_b, fc1_W, fc1_b, fc2_W, fc2_b, fc3_W, fc3_b,
                  out_ref, ebuf0, ebuf1, sems):
    (ne0_Wt, ne0_b, ne1_W, ne1_b, g1_W, g1_b, g2_Wt, g2_b,
     g3a_W, g3a_b, g3b_W, g3b_b, ce0_Wt, ce0_b, ce1_W, ce1_b,
     fc0_Wt, fc0_b, fc1_W, fc1_b, fc2_W, fc2_b, fc3_W) = (
        r[...] for r in (
            ne0_Wt, ne0_b, ne1_W, ne1_b, g1_W, g1_b, g2_Wt, g2_b,
            g3a_W, g3a_b, g3b_W, g3b_b, ce0_Wt, ce0_b, ce1_W, ce1_b,
            fc0_Wt, fc0_b, fc1_W, fc1_b, fc2_W, fc2_b, fc3_W))
    bufs = (ebuf0, ebuf1)

    def copy(g):
        return pltpu.make_async_copy(
            edges_hbm.at[pl.ds(g * ROWS, ROWS), :],
            bufs[g % 2], sems.at[g % 2])

    copy(0).start()

    W_ne = _dot(ne1_W, ne0_Wt, _NT)
    b_ne = _dot(ne0_b, ne1_W, _NT) + ne1_b
    g1n = g1_W[:, :8]
    g1h = g1_W[:, 8:]
    Wp = _dot(g1h, W_ne, _NN)
    b0 = _dot(b_ne, g1h, _NT) + g1_b

    n0s = []
    for b in range(B):
        Xn = nodesT_ref[b * 8:(b + 1) * 8, :]
        Xp = nparamsT_ref[b * 16:(b + 1) * 16, :]
        n0s.append(_dot(Xn, g1n, _TT) + _dot(Xp, Wp, _TT) + b0)

    mus = []
    for g in range(NG):
        if g + 1 < NG:
            copy(g + 1).start()
        copy(g).wait()
        ebuf = bufs[g % 2]
        A_grp = (ebuf[1::C, :] + ebuf[2::C, :]
                 + ebuf[3::C, :])
        deg_grp = jnp.sum(A_grp, axis=1, keepdims=True) + 1e-8

        As = [A_grp[j * V:(j + 1) * V, :] for j in range(GR)]
        degs = [deg_grp[j * V:(j + 1) * V, :] for j in range(GR)]

        m1_parts = [
            _dot(As[j], n0s[g * GR + j], _NN) / degs[j] for j in range(GR)]
        m1 = jnp.concatenate(m1_parts, axis=0)
        n1 = _dot(m1, g2_Wt, _NN) + g2_b

        m2s = [_dot(As[j], n1[j * V:(j + 1) * V], _NN) / degs[j]
               for j in range(GR)]
        mean_m2s = [jnp.mean(m2s[j], axis=0, keepdims=True)
                    for j in range(GR)]
        ds = [m2s[j] - mean_m2s[j] for j in range(GR)]
        ads = [_dot(As[j], ds[j], _NN) for j in range(GR)]
        for j in range(GR):
            degb = degs[j]
            ev = ((degb - 1e-8) * ds[j] - ads[j]) / degb
            mus.append(jnp.concatenate(
                [mean_m2s[j],
                 jnp.mean(ev, axis=0, keepdims=True)], axis=1))
    mu = jnp.concatenate(mus, axis=0)

    gl = _dot(mu, g3a_W, _NT) + g3a_b
    gl = _dot(gl, g3b_W, _NT) + g3b_b
    c = _dot(condT_ref[...], ce0_Wt, _TN) + ce0_b
    c = _dot(c, ce1_W, _NT) + ce1_b
    gl = jnp.concatenate([gl, c], axis=1)
    gl = _dot(gl, fc0_Wt, _NN) + fc0_b
    gl = _dot(gl, fc1_W, _NT) + fc1_b
    gl = _dot(gl, fc2_W, _NT) + fc2_b
    t = gl * fc3_W
    s = _dot(t, jnp.ones((32, V), _F32), _NN)
    out_ref[...] = s + fc3_b[0, 0]


@functools.partial(jax.jit, static_argnames=())
def kernel(edges, hidden, nodes, node_params, cond,
           ne0_W, ne0_b, ne1_W, ne1_b, g1_W, g1_b, g2_W, g2_b,
           g3a_W, g3a_b, g3b_W, g3b_b, ce0_W, ce0_b, ce1_W, ce1_b,
           fc0_W, fc0_b, fc1_W, fc1_b, fc2_W, fc2_b, fc3_W, fc3_b):
    del hidden
    edges2d = edges.transpose(0, 1, 3, 2).reshape(B * V * C, V)
    nodesT = nodes.transpose(0, 2, 1).reshape(B * 8, V)
    nparamsT = node_params.transpose(0, 2, 1).reshape(B * 16, V)
    args = [edges2d, nodesT, nparamsT, cond.T,
            ne0_W.T, ne0_b.reshape(1, -1), ne1_W, ne1_b.reshape(1, -1),
            g1_W, g1_b.reshape(1, -1), g2_W.T, g2_b.reshape(1, -1),
            g3a_W, g3a_b.reshape(1, -1), g3b_W, g3b_b.reshape(1, -1),
            ce0_W.T, ce0_b.reshape(1, -1), ce1_W, ce1_b.reshape(1, -1),
            fc0_W.T, fc0_b.reshape(1, -1), fc1_W, fc1_b.reshape(1, -1),
            fc2_W, fc2_b.reshape(1, -1), fc3_W, fc3_b.reshape(1, -1)]
    in_specs = [pl.BlockSpec(memory_space=pltpu.MemorySpace.HBM)]
    in_specs += [pl.BlockSpec(memory_space=pltpu.MemorySpace.VMEM)
                 for _ in range(len(args) - 1)]
    out = pl.pallas_call(
        _fused_kernel,
        in_specs=in_specs,
        out_specs=pl.BlockSpec(memory_space=pltpu.MemorySpace.VMEM),
        out_shape=jax.ShapeDtypeStruct((B, V), jnp.float32),
        scratch_shapes=[pltpu.VMEM((ROWS, V), jnp.float32),
                        pltpu.VMEM((ROWS, V), jnp.float32),
                        pltpu.SemaphoreType.DMA((2,))],
    )(*args)
    return out[:, :1]
